# baseline (speedup 1.0000x reference)
import jax
import jax.numpy as jnp
from jax import lax
from jax.experimental import pallas as pl
from jax.experimental.pallas import tpu as pltpu
from jax.experimental.pallas import tpu_sc as plsc

NC = 2
NS = 16
NW = NC * NS


def _sc_aggregate(x, srcw, dstw, n_pad, n_cnt):
  d = x.shape[1]
  _, nchunks, C = srcw.shape
  rows_per_sub = n_pad // NS
  cnt_per_sub = n_cnt // NS

  zeros_blk = jnp.zeros((rows_per_sub, d), jnp.float32)
  zeros_cnt = jnp.zeros((cnt_per_sub,), jnp.float32)
  ones_blk = jnp.ones((C,), jnp.float32)

  mesh = plsc.VectorSubcoreMesh(core_axis_name="c", subcore_axis_name="s",
                                num_cores=NC, num_subcores=NS)

  assert nchunks >= 3 and (nchunks - 3) % 2 == 0
  loop_iters = (nchunks - 3) // 2

  def body(x_hbm, src_hbm, dst_hbm, zf_hbm, zc_hbm, ones_hbm,
           sum_out, cnt0_out, cnt1_out,
           acc, cnt_sh, src_v, dst_v, rows0, rows1, ones_v,
           g0, g1, s0, s1, isem):
    cid = lax.axis_index("c")
    sid = lax.axis_index("s")
    wid = cid * NS + sid
    base_n = sid * rows_per_sub
    bufs = ((rows0, g0, s0), (rows1, g1, s1))

    def start_gather(i, b):
      rv, g, _ = bufs[b]
      pltpu.async_copy(x_hbm.at[src_v.at[i]], rv, g)

    def wait_gather(i, b):
      rv, g, _ = bufs[b]
      pltpu.make_async_copy(x_hbm.at[src_v.at[i]], rv, g).wait()

    def start_scatter(i, b):
      rv, _, s = bufs[b]
      pltpu.async_copy(rv, acc.at[dst_v.at[i]], s, add=True)
      pltpu.async_copy(ones_v, cnt_sh.at[dst_v.at[i]], s, add=True)

    def wait_scatter(i, b):
      rv, _, s = bufs[b]
      pltpu.make_async_copy(rv, acc.at[dst_v.at[i]], s).wait()
      pltpu.make_async_copy(ones_v, cnt_sh.at[dst_v.at[i]], s).wait()

    pltpu.sync_copy(src_hbm.at[wid], src_v)
    pltpu.sync_copy(dst_hbm.at[wid], dst_v)
    pltpu.sync_copy(ones_hbm, ones_v)
    start_gather(0, 0)
    start_gather(1, 1)
    pltpu.sync_copy(zf_hbm, acc.at[pl.ds(base_n, rows_per_sub)])
    pltpu.sync_copy(zc_hbm, cnt_sh.at[pl.ds(sid * cnt_per_sub, cnt_per_sub)])
    plsc.subcore_barrier()
    wait_gather(0, 0)
    start_scatter(0, 0)

    def chunk2(k, carry):
      i0 = 2 + 2 * k
      wait_scatter(i0 - 2, 0)
      start_gather(i0, 0)
      wait_gather(i0 - 1, 1)
      start_scatter(i0 - 1, 1)
      wait_scatter(i0 - 1, 1)
      start_gather(i0 + 1, 1)
      wait_gather(i0, 0)
      start_scatter(i0, 0)
      return carry

    lax.fori_loop(0, loop_iters, chunk2, 0)
    wait_scatter(nchunks - 3, 0)
    start_gather(nchunks - 1, 0)
    wait_gather(nchunks - 2, 1)
    start_scatter(nchunks - 2, 1)
    wait_gather(nchunks - 1, 0)
    start_scatter(nchunks - 1, 0)
    wait_scatter(nchunks - 2, 1)
    wait_scatter(nchunks - 1, 0)
    plsc.subcore_barrier()
    pltpu.sync_copy(acc.at[pl.ds(base_n, rows_per_sub)],
                    sum_out.at[cid, pl.ds(base_n, rows_per_sub)])

    @pl.when(cid == 0)
    def _():
      pltpu.sync_copy(cnt_sh.at[pl.ds(sid * cnt_per_sub, cnt_per_sub)],
                      cnt0_out.at[pl.ds(sid * cnt_per_sub, cnt_per_sub)])

    @pl.when(cid == 1)
    def _():
      pltpu.sync_copy(cnt_sh.at[pl.ds(sid * cnt_per_sub, cnt_per_sub)],
                      cnt1_out.at[pl.ds(sid * cnt_per_sub, cnt_per_sub)])

  call = pl.kernel(
      body,
      out_type=(
          jax.ShapeDtypeStruct((NC, n_pad, d), jnp.float32),
          jax.ShapeDtypeStruct((n_cnt,), jnp.float32),
          jax.ShapeDtypeStruct((n_cnt,), jnp.float32),
      ),
      mesh=mesh,
      scratch_types=[
          pltpu.VMEM_SHARED((n_pad, d), jnp.float32),
          pltpu.VMEM_SHARED((n_cnt,), jnp.float32),
          pltpu.VMEM((nchunks, C), jnp.int32),
          pltpu.VMEM((nchunks, C), jnp.int32),
          pltpu.VMEM((C, d), jnp.float32),
          pltpu.VMEM((C, d), jnp.float32),
          pltpu.VMEM((C,), jnp.float32),
          pltpu.SemaphoreType.DMA,
          pltpu.SemaphoreType.DMA,
          pltpu.SemaphoreType.DMA,
          pltpu.SemaphoreType.DMA,
          pltpu.SemaphoreType.DMA,
      ],
      compiler_params=pltpu.CompilerParams(use_tc_tiling_on_sc=False),
  )
  return call(x, srcw, dstw, zeros_blk, zeros_cnt, ones_blk)


def _tc_mlp(x, s0, s1, counts, W_lT, W_rT, W1T, W2T, W3T, b_l, b1, b2, b3):
  n, d = x.shape
  out_dim = W3T.shape[1]
  R = 1000
  assert n % R == 0
  grid = n // R

  def body(xb, s0b, s1b, cb, wl, wr, w1, w2, w3, bl, bb1, bb2, bb3, ob):
    summed = s0b[...] + s1b[...]
    mean = summed / jnp.maximum(cb[...], 1.0)
    h = (jnp.dot(mean, wl[...], preferred_element_type=jnp.float32)
         + jnp.dot(xb[...], wr[...], preferred_element_type=jnp.float32)
         + bl[...])
    h1 = jnp.maximum(
        jnp.dot(h, w1[...], preferred_element_type=jnp.float32) + bb1[...], 0.0)
    h2 = jnp.maximum(
        jnp.dot(h1, w2[...], preferred_element_type=jnp.float32) + bb2[...], 0.0)
    ob[...] = jnp.dot(h2, w3[...], preferred_element_type=jnp.float32) + bb3[...]

  row_spec = lambda c: pl.BlockSpec((R, c), lambda i: (i, 0))
  full_spec = lambda r, c: pl.BlockSpec((r, c), lambda i: (0, 0))
  return pl.pallas_call(
      body,
      grid=(grid,),
      in_specs=[
          row_spec(d), row_spec(d), row_spec(d), row_spec(1),
          full_spec(*W_lT.shape), full_spec(*W_rT.shape),
          full_spec(*W1T.shape), full_spec(*W2T.shape), full_spec(*W3T.shape),
          full_spec(*b_l.shape), full_spec(*b1.shape),
          full_spec(*b2.shape), full_spec(*b3.shape),
      ],
      out_specs=row_spec(out_dim),
      out_shape=jax.ShapeDtypeStruct((n, out_dim), jnp.float32),
  )(x, s0, s1, counts, W_lT, W_rT, W1T, W2T, W3T, b_l, b1, b2, b3)


@jax.jit
def kernel(x, edge_index, W_l, b_l, W_r, W1, b1, W2, b2, W3, b3):
  n, d = x.shape
  e = edge_index.shape[1]
  n_pad = ((n + NS * 8 - 1) // (NS * 8)) * (NS * 8)
  n_cnt = ((n + NS * 8 - 1) // (NS * 8)) * (NS * 8)
  C = 96
  assert e % NW == 0
  e_per_w = e // NW
  nchunks = (e_per_w + C - 1) // C
  if nchunks % 2 == 0:
    nchunks += 1
  pad = nchunks * C - e_per_w
  ew = edge_index.reshape(2, NW, e_per_w)
  src_pad = jnp.zeros((NW, pad), jnp.int32)
  dst_pad = jnp.full((NW, pad), n, jnp.int32)
  srcw = jnp.concatenate([ew[0], src_pad], 1).reshape(NW, nchunks, C)
  dstw = jnp.concatenate([ew[1], dst_pad], 1).reshape(NW, nchunks, C)
  sums, cnt0, cnt1 = _sc_aggregate(x, srcw, dstw, n_pad, n_cnt)
  counts = (cnt0 + cnt1)[:n].reshape(n, 1)
  return _tc_mlp(
      x, sums[0], sums[1], counts,
      W_l.T, W_r.T, W1.T, W2.T, W3.T,
      b_l.reshape(1, -1), b1.reshape(1, -1), b2.reshape(1, -1),
      b3.reshape(1, -1))


# speedup vs baseline: 1.3081x; 1.3081x over previous
import jax
import jax.numpy as jnp
from jax import lax
from jax.experimental import pallas as pl
from jax.experimental.pallas import tpu as pltpu
from jax.experimental.pallas import tpu_sc as plsc

NC = 2
NS = 16
NW = NC * NS


def _sc_aggregate(x, src, dst, n_pad, n_cnt):
  e = src.shape[0]
  d = x.shape[1]
  assert e % NW == 0
  e_per_w = e // NW
  C = 80
  assert e_per_w % C == 0
  nchunks = e_per_w // C
  rows_per_sub = n_pad // NS
  cnt_per_sub = n_cnt // NS
  NB = 4

  zeros_blk = jnp.zeros((rows_per_sub, d), jnp.float32)
  zeros_cnt = jnp.zeros((cnt_per_sub,), jnp.float32)
  ones_blk = jnp.ones((C,), jnp.float32)

  mesh = plsc.VectorSubcoreMesh(core_axis_name="c", subcore_axis_name="s",
                                num_cores=NC, num_subcores=NS)

  assert nchunks > 2 * NB and (nchunks - NB - 1) % NB == 0
  loop_iters = (nchunks - NB - 1) // NB

  def body(x_hbm, src_hbm, dst_hbm, zf_hbm, zc_hbm, ones_hbm,
           sum_out, cnt0_out, cnt1_out, acc, cnt_sh, ones_v, *bufs):
    cid = lax.axis_index("c")
    sid = lax.axis_index("s")
    wid = cid * NS + sid
    base_n = sid * rows_per_sub
    base_e = wid * e_per_w
    svs, dvs, rvs, gs, ss = (bufs[0:NB], bufs[NB:2 * NB], bufs[2 * NB:3 * NB],
                             bufs[3 * NB:4 * NB], bufs[4 * NB:5 * NB])

    def start_chunk(i, b):
      off = base_e + i * C
      pltpu.sync_copy(src_hbm.at[pl.ds(off, C)], svs[b])
      pltpu.sync_copy(dst_hbm.at[pl.ds(off, C)], dvs[b])
      pltpu.async_copy(x_hbm.at[svs[b]], rvs[b], gs[b])

    def wait_gather(b):
      pltpu.make_async_copy(x_hbm.at[svs[b]], rvs[b], gs[b]).wait()

    def start_scatter(b):
      pltpu.async_copy(rvs[b], acc.at[dvs[b]], ss[b], add=True)
      pltpu.async_copy(ones_v, cnt_sh.at[dvs[b]], ss[b], add=True)

    def wait_scatter(b):
      pltpu.make_async_copy(rvs[b], acc.at[dvs[b]], ss[b]).wait()
      pltpu.make_async_copy(ones_v, cnt_sh.at[dvs[b]], ss[b]).wait()

    pltpu.sync_copy(ones_hbm, ones_v)
    for b in range(NB):
      start_chunk(b, b)
    pltpu.sync_copy(zf_hbm, acc.at[pl.ds(base_n, rows_per_sub)])
    pltpu.sync_copy(zc_hbm, cnt_sh.at[pl.ds(sid * cnt_per_sub, cnt_per_sub)])
    plsc.subcore_barrier()
    for b in range(NB - 1):
      wait_gather(b)
      start_scatter(b)

    def chunk_group(k, carry):
      i0 = NB + NB * k
      for b in range(NB):
        wait_scatter(b)
        start_chunk(i0 + b, b)
        wait_gather((b - 1) % NB)
        start_scatter((b - 1) % NB)
      return carry

    lax.fori_loop(0, loop_iters, chunk_group, 0)
    last = nchunks - 1
    b = last % NB
    wait_scatter(b)
    start_chunk(last, b)
    wait_gather((b - 1) % NB)
    start_scatter((b - 1) % NB)
    wait_gather(b)
    start_scatter(b)
    for bb in range(NB):
      wait_scatter(bb)
    plsc.subcore_barrier()
    pltpu.sync_copy(acc.at[pl.ds(base_n, rows_per_sub)],
                    sum_out.at[cid, pl.ds(base_n, rows_per_sub)])

    @pl.when(cid == 0)
    def _():
      pltpu.sync_copy(cnt_sh.at[pl.ds(sid * cnt_per_sub, cnt_per_sub)],
                      cnt0_out.at[pl.ds(sid * cnt_per_sub, cnt_per_sub)])

    @pl.when(cid == 1)
    def _():
      pltpu.sync_copy(cnt_sh.at[pl.ds(sid * cnt_per_sub, cnt_per_sub)],
                      cnt1_out.at[pl.ds(sid * cnt_per_sub, cnt_per_sub)])

  call = pl.kernel(
      body,
      out_type=(
          jax.ShapeDtypeStruct((NC, n_pad, d), jnp.float32),
          jax.ShapeDtypeStruct((n_cnt,), jnp.float32),
          jax.ShapeDtypeStruct((n_cnt,), jnp.float32),
      ),
      mesh=mesh,
      scratch_types=(
          [
              pltpu.VMEM_SHARED((n_pad, d), jnp.float32),
              pltpu.VMEM_SHARED((n_cnt,), jnp.float32),
              pltpu.VMEM((C,), jnp.float32),
          ]
          + [pltpu.VMEM((C,), jnp.int32) for _ in range(2 * NB)]
          + [pltpu.VMEM((C, d), jnp.float32) for _ in range(NB)]
          + [pltpu.SemaphoreType.DMA for _ in range(2 * NB)]
      ),
      compiler_params=pltpu.CompilerParams(use_tc_tiling_on_sc=False),
  )
  return call(x, src, dst, zeros_blk, zeros_cnt, ones_blk)


def _tc_mlp(x, s0, s1, counts, W_lT, W_rT, W1T, W2T, W3T, b_l, b1, b2, b3):
  n, d = x.shape
  out_dim = W3T.shape[1]
  R = 1000
  assert n % R == 0
  grid = n // R

  def body(xb, s0b, s1b, cb, wl, wr, w1, w2, w3, bl, bb1, bb2, bb3, ob):
    summed = s0b[...] + s1b[...]
    mean = summed / jnp.maximum(cb[...], 1.0)
    h = (jnp.dot(mean, wl[...], preferred_element_type=jnp.float32)
         + jnp.dot(xb[...], wr[...], preferred_element_type=jnp.float32)
         + bl[...])
    h1 = jnp.maximum(
        jnp.dot(h, w1[...], preferred_element_type=jnp.float32) + bb1[...], 0.0)
    h2 = jnp.maximum(
        jnp.dot(h1, w2[...], preferred_element_type=jnp.float32) + bb2[...], 0.0)
    ob[...] = jnp.dot(h2, w3[...], preferred_element_type=jnp.float32) + bb3[...]

  row_spec = lambda c: pl.BlockSpec((R, c), lambda i: (i, 0))
  full_spec = lambda r, c: pl.BlockSpec((r, c), lambda i: (0, 0))
  return pl.pallas_call(
      body,
      grid=(grid,),
      in_specs=[
          row_spec(d), row_spec(d), row_spec(d), row_spec(1),
          full_spec(*W_lT.shape), full_spec(*W_rT.shape),
          full_spec(*W1T.shape), full_spec(*W2T.shape), full_spec(*W3T.shape),
          full_spec(*b_l.shape), full_spec(*b1.shape),
          full_spec(*b2.shape), full_spec(*b3.shape),
      ],
      out_specs=row_spec(out_dim),
      out_shape=jax.ShapeDtypeStruct((n, out_dim), jnp.float32),
  )(x, s0, s1, counts, W_lT, W_rT, W1T, W2T, W3T, b_l, b1, b2, b3)


@jax.jit
def kernel(x, edge_index, W_l, b_l, W_r, W1, b1, W2, b2, W3, b3):
  n, d = x.shape
  src = edge_index[0]
  dst = edge_index[1]
  n_pad = ((n + NS * 8 - 1) // (NS * 8)) * (NS * 8)
  n_cnt = ((n + NS * 8 - 1) // (NS * 8)) * (NS * 8)
  sums, cnt0, cnt1 = _sc_aggregate(x, src, dst, n_pad, n_cnt)
  counts = (cnt0 + cnt1)[:n].reshape(n, 1)
  return _tc_mlp(
      x, sums[0], sums[1], counts,
      W_l.T, W_r.T, W1.T, W2.T, W3.T,
      b_l.reshape(1, -1), b1.reshape(1, -1), b2.reshape(1, -1),
      b3.reshape(1, -1))


# speedup vs baseline: 1.3727x; 1.0494x over previous
import jax
import jax.numpy as jnp
from jax import lax
from jax.experimental import pallas as pl
from jax.experimental.pallas import tpu as pltpu
from jax.experimental.pallas import tpu_sc as plsc

NC = 2
NS = 16
NW = NC * NS
CW = 8


def _sc_aggregate(x, edge_index, n_pad):
  e = edge_index.shape[1]
  d = x.shape[1]
  assert e % NW == 0
  e_per_w = e // NW
  C = 80
  assert e_per_w % C == 0
  nchunks = e_per_w // C
  rows_per_sub = n_pad // NS
  NB = 4

  zeros_blk = jnp.zeros((rows_per_sub, d), jnp.float32)
  zeros_cnt = jnp.zeros((rows_per_sub, CW), jnp.float32)
  ones_blk = jnp.ones((C, CW), jnp.float32)

  mesh = plsc.VectorSubcoreMesh(core_axis_name="c", subcore_axis_name="s",
                                num_cores=NC, num_subcores=NS)

  assert nchunks > 2 * NB and (nchunks - NB - 1) % NB == 0
  loop_iters = (nchunks - NB - 1) // NB

  def body(x_hbm, ei_hbm, zf_hbm, zc_hbm, ones_hbm,
           sum_out, cnt0_out, cnt1_out, acc, cnt_sh, ones_v, *bufs):
    cid = lax.axis_index("c")
    sid = lax.axis_index("s")
    wid = cid * NS + sid
    base_n = sid * rows_per_sub
    base_e = wid * e_per_w
    svs, dvs, rvs, gs, ss = (bufs[0:NB], bufs[NB:2 * NB], bufs[2 * NB:3 * NB],
                             bufs[3 * NB:4 * NB], bufs[4 * NB:5 * NB])

    def start_chunk(i, b):
      off = base_e + i * C
      pltpu.sync_copy(ei_hbm.at[0, pl.ds(off, C)], svs[b])
      pltpu.sync_copy(ei_hbm.at[1, pl.ds(off, C)], dvs[b])
      pltpu.async_copy(x_hbm.at[svs[b]], rvs[b], gs[b])

    def wait_gather(b):
      pltpu.make_async_copy(x_hbm.at[svs[b]], rvs[b], gs[b]).wait()

    def start_scatter(b):
      pltpu.async_copy(rvs[b], acc.at[dvs[b]], ss[b], add=True)
      pltpu.async_copy(ones_v, cnt_sh.at[dvs[b]], ss[b], add=True)

    def wait_scatter(b):
      pltpu.make_async_copy(rvs[b], acc.at[dvs[b]], ss[b]).wait()
      pltpu.make_async_copy(ones_v, cnt_sh.at[dvs[b]], ss[b]).wait()

    pltpu.sync_copy(ones_hbm, ones_v)
    for b in range(NB):
      start_chunk(b, b)
    pltpu.sync_copy(zf_hbm, acc.at[pl.ds(base_n, rows_per_sub)])
    pltpu.sync_copy(zc_hbm, cnt_sh.at[pl.ds(base_n, rows_per_sub)])
    plsc.subcore_barrier()
    for b in range(NB - 1):
      wait_gather(b)
      start_scatter(b)

    def chunk_group(k, carry):
      i0 = NB + NB * k
      for b in range(NB):
        wait_scatter(b)
        start_chunk(i0 + b, b)
        wait_gather((b - 1) % NB)
        start_scatter((b - 1) % NB)
      return carry

    lax.fori_loop(0, loop_iters, chunk_group, 0)
    last = nchunks - 1
    b = last % NB
    wait_scatter(b)
    start_chunk(last, b)
    wait_gather((b - 1) % NB)
    start_scatter((b - 1) % NB)
    wait_gather(b)
    start_scatter(b)
    for bb in range(NB):
      wait_scatter(bb)
    plsc.subcore_barrier()
    pltpu.sync_copy(acc.at[pl.ds(base_n, rows_per_sub)],
                    sum_out.at[cid, pl.ds(base_n, rows_per_sub)])

    @pl.when(cid == 0)
    def _():
      pltpu.sync_copy(cnt_sh.at[pl.ds(base_n, rows_per_sub)],
                      cnt0_out.at[pl.ds(base_n, rows_per_sub)])

    @pl.when(cid == 1)
    def _():
      pltpu.sync_copy(cnt_sh.at[pl.ds(base_n, rows_per_sub)],
                      cnt1_out.at[pl.ds(base_n, rows_per_sub)])

  call = pl.kernel(
      body,
      out_type=(
          jax.ShapeDtypeStruct((NC, n_pad, d), jnp.float32),
          jax.ShapeDtypeStruct((n_pad, CW), jnp.float32),
          jax.ShapeDtypeStruct((n_pad, CW), jnp.float32),
      ),
      mesh=mesh,
      scratch_types=(
          [
              pltpu.VMEM_SHARED((n_pad, d), jnp.float32),
              pltpu.VMEM_SHARED((n_pad, CW), jnp.float32),
              pltpu.VMEM((C, CW), jnp.float32),
          ]
          + [pltpu.VMEM((C,), jnp.int32) for _ in range(2 * NB)]
          + [pltpu.VMEM((C, d), jnp.float32) for _ in range(NB)]
          + [pltpu.SemaphoreType.DMA for _ in range(2 * NB)]
      ),
      compiler_params=pltpu.CompilerParams(use_tc_tiling_on_sc=False),
  )
  return call(x, edge_index, zeros_blk, zeros_cnt, ones_blk)


def _tc_mlp(x, sums, cnt0, cnt1, W_l, W_r, W1, W2, W3, b_l, b1, b2, b3):
  n, d = x.shape
  out_dim = W3.shape[0]
  R = 1000
  assert n % R == 0
  grid = n // R
  dn = (((1,), (1,)), ((), ()))

  def body(xb, s0b, s1b, c0b, c1b, wl, wr, w1, w2, w3,
           bl, bb1, bb2, bb3, ob):
    summed = s0b[0] + s1b[0]
    counts = c0b[:, :1] + c1b[:, :1]
    mean = summed / jnp.maximum(counts, 1.0)
    f32 = jnp.float32
    h = (lax.dot_general(mean, wl[...], dn, preferred_element_type=f32)
         + lax.dot_general(xb[...], wr[...], dn, preferred_element_type=f32)
         + bl[...])
    h1 = jnp.maximum(
        lax.dot_general(h, w1[...], dn, preferred_element_type=f32)
        + bb1[...], 0.0)
    h2 = jnp.maximum(
        lax.dot_general(h1, w2[...], dn, preferred_element_type=f32)
        + bb2[...], 0.0)
    ob[...] = (lax.dot_general(h2, w3[...], dn, preferred_element_type=f32)
               + bb3[...])

  row_spec = lambda c: pl.BlockSpec((R, c), lambda i: (i, 0))
  sum_spec = lambda k: pl.BlockSpec((1, R, d), lambda i, _k=k: (_k, i, 0))
  full_spec = lambda r, c: pl.BlockSpec((r, c), lambda i: (0, 0))
  return pl.pallas_call(
      body,
      grid=(grid,),
      in_specs=[
          row_spec(d), sum_spec(0), sum_spec(1), row_spec(CW), row_spec(CW),
          full_spec(*W_l.shape), full_spec(*W_r.shape),
          full_spec(*W1.shape), full_spec(*W2.shape), full_spec(*W3.shape),
          full_spec(*b_l.shape), full_spec(*b1.shape),
          full_spec(*b2.shape), full_spec(*b3.shape),
      ],
      out_specs=row_spec(out_dim),
      out_shape=jax.ShapeDtypeStruct((n, out_dim), jnp.float32),
  )(x, sums, sums, cnt0, cnt1, W_l, W_r, W1, W2, W3, b_l, b1, b2, b3)


@jax.jit
def kernel(x, edge_index, W_l, b_l, W_r, W1, b1, W2, b2, W3, b3):
  n, d = x.shape
  n_pad = ((n + NS * 8 - 1) // (NS * 8)) * (NS * 8)
  sums, cnt0, cnt1 = _sc_aggregate(x, edge_index, n_pad)
  return _tc_mlp(
      x, sums, cnt0, cnt1, W_l, W_r, W1, W2, W3,
      b_l.reshape(1, -1), b1.reshape(1, -1), b2.reshape(1, -1),
      b3.reshape(1, -1))
